# Initial kernel scaffold; baseline (speedup 1.0000x reference)
import jax
import jax.numpy as jnp
from jax.experimental import pallas as pl


def kernel(x):
    raise NotImplementedError("write your pallas kernel here")


# speedup vs baseline: 10.1095x; 10.1095x over previous
import jax
import jax.numpy as jnp
import numpy as np
from jax import lax
from jax.experimental import pallas as pl
from jax.experimental.pallas import tpu as pltpu
from jax.experimental.pallas import tpu_sc as plsc

K = 256
NCORES = 2
NSUB = 16
NW = NCORES * NSUB
ROWS = 64
ROWS_PER_W = ROWS // NW
ROW_ELEMS = 32 * 32768
CHUNK = 16384
NCHUNK = ROW_ELEMS // CHUNK
SUBS = 8
VECS_PER_SUB = CHUNK // (SUBS * 16)
TRIG = 2047
CAPB = TRIG + 1 + CHUNK

_SIGN = np.uint32(0x80000000)


def _keys16(xv):
    b = lax.bitcast_convert_type(xv, jnp.uint32)
    return jnp.where(b >= _SIGN, ~b, b | _SIGN)


def _unkey16(kv):
    b = jnp.where(kv >= _SIGN, kv & np.uint32(0x7FFFFFFF), ~kv)
    return lax.bitcast_convert_type(b, jnp.float32)


def _mk_kernel():
    mesh = plsc.VectorSubcoreMesh(
        core_axis_name="c", subcore_axis_name="s", num_cores=NCORES)

    def body(x_hbm, out_hbm, chunk, buf, keybuf, topbuf, outv, thr_ref, ptr_ref):
        cid = lax.axis_index("c")
        sid = lax.axis_index("s")
        wid = sid * NCORES + cid
        lanes = lax.iota(jnp.int32, 16)

        def fill_keys(nvec, count):
            def kb(j, _):
                xv = buf[pl.ds(j * 16, 16)]
                valid = (j * 16 + lanes) < count
                kv = jnp.where(valid, _keys16(xv), np.uint32(0))
                keybuf[pl.ds(j * 16, 16)] = kv
                return 0
            lax.fori_loop(0, nvec, kb, 0)

        def kth_key(nvec):
            def bit_step(b, acc):
                t = acc | (np.uint32(1) << (np.uint32(31) - b.astype(jnp.uint32)))
                def cstep(j, cv):
                    kv = keybuf[pl.ds(j * 16, 16)]
                    return cv + (kv >= t).astype(jnp.int32)
                cnt = jnp.sum(lax.fori_loop(0, nvec, cstep, jnp.zeros((16,), jnp.int32)))
                return jnp.where(cnt >= K, t, acc)
            return lax.fori_loop(0, 32, bit_step, jnp.zeros((), jnp.uint32))

        def strict_stats(nvec, tkey):
            def sstep(j, cs):
                cv, sv = cs
                kv = keybuf[pl.ds(j * 16, 16)]
                xv = buf[pl.ds(j * 16, 16)]
                m = kv > tkey
                return (cv + m.astype(jnp.int32), sv + jnp.where(m, xv, 0.0))
            cv, sv = lax.fori_loop(0, nvec, sstep,
                                   (jnp.zeros((16,), jnp.int32),
                                    jnp.zeros((16,), jnp.float32)))
            return jnp.sum(cv), jnp.sum(sv)

        def compact():
            count = ptr_ref[0]
            nvec = (count + 15) >> 4
            fill_keys(nvec, count)
            tkey = kth_key(nvec)
            tvec = _unkey16(jnp.full((16,), tkey, jnp.uint32))
            tf = jnp.max(tvec)

            def cstep(j, np_):
                kv = keybuf[pl.ds(j * 16, 16)]
                xv = buf[pl.ds(j * 16, 16)]
                m = kv > tkey
                cs = plsc.cumsum(m.astype(jnp.int32))
                plsc.store_scatter(topbuf, [np_ + cs - 1], xv, mask=m)
                return np_ + jnp.max(cs)
            newptr = lax.fori_loop(0, nvec, cstep, 0)
            for j in range(K // 16):
                topbuf[pl.ds(newptr + j * 16, 16)] = tvec
            for j in range(K // 16):
                buf[pl.ds(j * 16, 16)] = topbuf[pl.ds(j * 16, 16)]
            ptr_ref[0] = K
            thr_ref[0] = tf

        for r in range(ROWS_PER_W):
            row = wid * ROWS_PER_W + r
            thr_ref[0] = jnp.full((), -np.inf, jnp.float32)
            ptr_ref[0] = 0

            def chunk_step(i, _):
                pltpu.sync_copy(x_hbm.at[row * NCHUNK + i], chunk)

                def sub_step(s, _s):
                    def vstep(v, _v):
                        xv = chunk[pl.ds((s * VECS_PER_SUB + v) * 16, 16)]
                        m = xv > thr_ref[0]
                        p = ptr_ref[0]
                        cs = plsc.cumsum(m.astype(jnp.int32))
                        plsc.store_scatter(buf, [p + cs - 1], xv, mask=m)
                        ptr_ref[0] = p + jnp.max(cs)
                        return 0
                    lax.fori_loop(0, VECS_PER_SUB, vstep, 0)

                    @pl.when(ptr_ref[0] > TRIG)
                    def _():
                        compact()
                    return 0
                lax.fori_loop(0, SUBS, sub_step, 0)
                return 0
            lax.fori_loop(0, NCHUNK, chunk_step, 0)

            count = ptr_ref[0]
            nvec = (count + 15) >> 4
            fill_keys(nvec, count)
            tkey = kth_key(nvec)
            tvec = _unkey16(jnp.full((16,), tkey, jnp.uint32))
            tf = jnp.max(tvec)
            c, s = strict_stats(nvec, tkey)
            mean = (s + (K - c).astype(jnp.float32) * tf) * np.float32(1.0 / K)
            if r == 0:
                outv[...] = jnp.where(lanes == 0, mean, 0.0)
            else:
                outv[...] = jnp.where(lanes == r, mean, outv[...])
        pltpu.sync_copy(outv, out_hbm.at[wid])

    return pl.kernel(
        body,
        mesh=mesh,
        compiler_params=pltpu.CompilerParams(needs_layout_passes=False),
        out_type=jax.ShapeDtypeStruct((NW, 16), jnp.float32),
        scratch_types=[
            pltpu.VMEM((CHUNK,), jnp.float32),
            pltpu.VMEM((CAPB,), jnp.float32),
            pltpu.VMEM((CAPB,), jnp.uint32),
            pltpu.VMEM((2 * K,), jnp.float32),
            pltpu.VMEM((16,), jnp.float32),
            pltpu.SMEM((1,), jnp.float32),
            pltpu.SMEM((1,), jnp.int32),
        ],
    )


_kernel_call_cache = []


def kernel(x):
    if not _kernel_call_cache:
        _kernel_call_cache.append(_mk_kernel())
    xf = x.reshape(ROWS * NCHUNK, CHUNK)
    out = _kernel_call_cache[0](xf)
    return out[:, :ROWS_PER_W].reshape(ROWS)


# speedup vs baseline: 22.5080x; 2.2264x over previous
import jax
import jax.numpy as jnp
import numpy as np
from jax import lax
from jax.experimental import pallas as pl
from jax.experimental.pallas import tpu as pltpu
from jax.experimental.pallas import tpu_sc as plsc

K = 256
NCORES = 2
NSUB = 16
NW = NCORES * NSUB
ROWS = 64
ROWS_PER_W = ROWS // NW
ROW_ELEMS = 32 * 32768
CHUNK = 16384
NCHUNK = ROW_ELEMS // CHUNK
SUBVEC = 256
SUBS = CHUNK // (SUBVEC * 16)
UNROLL = 8
SUBCAP = 512
LTRIG = 255

_SIGN = np.uint32(0x80000000)


def _keys16(xv):
    b = lax.bitcast_convert_type(xv, jnp.uint32)
    return jnp.where(b >= _SIGN, ~b, b | _SIGN)


def _unkey16(kv):
    b = jnp.where(kv >= _SIGN, kv & np.uint32(0x7FFFFFFF), ~kv)
    return lax.bitcast_convert_type(b, jnp.float32)


def _mk_kernel():
    mesh = plsc.VectorSubcoreMesh(
        core_axis_name="c", subcore_axis_name="s", num_cores=NCORES)

    def body(x_hbm, out_hbm, chunk, buf, keybuf, outv, d_ref, tkey_ref):
        cid = lax.axis_index("c")
        sid = lax.axis_index("s")
        wid = sid * NCORES + cid
        lanes = lax.iota(jnp.int32, 16)

        def fill_keys(nvec, ptr_vec):
            def kb(j, _):
                xv = buf[pl.ds(j * 16, 16)]
                valid = j < ptr_vec
                kv = jnp.where(valid, _keys16(xv), np.uint32(0))
                keybuf[pl.ds(j * 16, 16)] = kv
                return 0
            lax.fori_loop(0, nvec, kb, 0)

        def kth_key(nvec):
            d = d_ref[0]
            tprev = tkey_ref[0]

            def bit_step(b, acc):
                t = acc | (np.uint32(1) << (np.uint32(31) - b.astype(jnp.uint32)))

                def cstep(j, cv):
                    kv = keybuf[pl.ds(j * 16, 16)]
                    return cv + (kv >= t).astype(jnp.int32)
                cv = lax.fori_loop(0, nvec, cstep, jnp.zeros((16,), jnp.int32))
                cnt = jnp.sum(cv) + jnp.where(tprev >= t, d, 0)
                return jnp.where(cnt >= K, t, acc)
            return lax.fori_loop(0, 32, bit_step, jnp.zeros((), jnp.uint32))

        def compact(ptr_vec, tv):
            nvec = jnp.max(ptr_vec)
            fill_keys(nvec, ptr_vec)
            tkey = kth_key(nvec)
            new_tv = _unkey16(jnp.full((16,), tkey, jnp.uint32))

            def cstep(j, newp):
                kv = keybuf[pl.ds(j * 16, 16)]
                xv = buf[pl.ds(j * 16, 16)]
                m = kv > tkey
                idx = jnp.left_shift(newp, 4) | lanes
                plsc.store_scatter(buf, [idx], xv, mask=m)
                return newp + m.astype(jnp.int32)
            newp = lax.fori_loop(0, nvec, cstep, jnp.zeros((16,), jnp.int32))
            d_ref[0] = K - jnp.sum(newp)
            tkey_ref[0] = tkey
            return newp, new_tv

        def maybe_compact(ptr_vec, tv):
            return lax.cond(jnp.max(ptr_vec) > LTRIG, compact,
                            lambda p, t: (p, t), ptr_vec, tv)

        for r in range(ROWS_PER_W):
            row = wid * ROWS_PER_W + r
            d_ref[0] = 0
            tkey_ref[0] = jnp.zeros((), jnp.uint32)
            state = (jnp.zeros((16,), jnp.int32),
                     jnp.full((16,), -np.inf, jnp.float32))

            def chunk_step(i, state):
                pltpu.sync_copy(x_hbm.at[row * NCHUNK + i], chunk)

                def sub_step(s, state):
                    def vstep(v, state):
                        ptr_vec, tv = state
                        for u in range(UNROLL):
                            xv = chunk[pl.ds((s * SUBVEC + v * UNROLL + u) * 16, 16)]
                            m = xv > tv
                            idx = jnp.left_shift(ptr_vec, 4) | lanes
                            plsc.store_scatter(buf, [idx], xv, mask=m)
                            ptr_vec = ptr_vec + m.astype(jnp.int32)
                        return (ptr_vec, tv)
                    state = lax.fori_loop(0, SUBVEC // UNROLL, vstep, state)
                    return maybe_compact(*state)
                return lax.fori_loop(0, SUBS, sub_step, state)
            ptr_vec, tv = lax.fori_loop(0, NCHUNK, chunk_step, state)

            nvec = jnp.max(ptr_vec)
            fill_keys(nvec, ptr_vec)
            tkey = kth_key(nvec)
            tf = jnp.max(_unkey16(jnp.full((16,), tkey, jnp.uint32)))

            def sstep(j, cs):
                cv, sv = cs
                kv = keybuf[pl.ds(j * 16, 16)]
                xv = buf[pl.ds(j * 16, 16)]
                m = kv > tkey
                return (cv + m.astype(jnp.int32), sv + jnp.where(m, xv, 0.0))
            cv, sv = lax.fori_loop(0, nvec, sstep,
                                   (jnp.zeros((16,), jnp.int32),
                                    jnp.zeros((16,), jnp.float32)))
            c = jnp.sum(cv)
            s = jnp.sum(sv)
            mean = (s + (K - c).astype(jnp.float32) * tf) * np.float32(1.0 / K)
            if r == 0:
                outv[...] = jnp.where(lanes == 0, mean, 0.0)
            else:
                outv[...] = jnp.where(lanes == r, mean, outv[...])
        pltpu.sync_copy(outv, out_hbm.at[wid])

    return pl.kernel(
        body,
        mesh=mesh,
        compiler_params=pltpu.CompilerParams(needs_layout_passes=False),
        out_type=jax.ShapeDtypeStruct((NW, 16), jnp.float32),
        scratch_types=[
            pltpu.VMEM((CHUNK,), jnp.float32),
            pltpu.VMEM((SUBCAP * 16,), jnp.float32),
            pltpu.VMEM((SUBCAP * 16,), jnp.uint32),
            pltpu.VMEM((16,), jnp.float32),
            pltpu.SMEM((1,), jnp.int32),
            pltpu.SMEM((1,), jnp.uint32),
        ],
    )


_kernel_call_cache = []


def kernel(x):
    if not _kernel_call_cache:
        _kernel_call_cache.append(_mk_kernel())
    xf = x.reshape(ROWS * NCHUNK, CHUNK)
    out = _kernel_call_cache[0](xf)
    return out[:, :ROWS_PER_W].reshape(ROWS)


# speedup vs baseline: 24.9924x; 1.1104x over previous
import jax
import jax.numpy as jnp
import numpy as np
from jax import lax
from jax.experimental import pallas as pl
from jax.experimental.pallas import tpu as pltpu
from jax.experimental.pallas import tpu_sc as plsc

K = 256
NCORES = 2
NSUB = 16
NW = NCORES * NSUB
ROWS = 64
ROWS_PER_W = ROWS // NW
ROW_ELEMS = 32 * 32768
CHUNK = 16384
NCHUNK = ROW_ELEMS // CHUNK
SUBVEC = 256
SUBS = CHUNK // (SUBVEC * 16)
UNROLL = 16
SUBCAP = 512
LTRIG = 255

_SIGN = np.uint32(0x80000000)


def _keys16(xv):
    b = lax.bitcast_convert_type(xv, jnp.uint32)
    return jnp.where(b >= _SIGN, ~b, b | _SIGN)


def _unkey16(kv):
    b = jnp.where(kv >= _SIGN, kv & np.uint32(0x7FFFFFFF), ~kv)
    return lax.bitcast_convert_type(b, jnp.float32)


def _mk_kernel():
    mesh = plsc.VectorSubcoreMesh(
        core_axis_name="c", subcore_axis_name="s", num_cores=NCORES)

    def body(x_hbm, out_hbm, chunks, buf, keybuf, outv, d_ref, tkey_ref, sems):
        cid = lax.axis_index("c")
        sid = lax.axis_index("s")
        wid = sid * NCORES + cid
        lanes = lax.iota(jnp.int32, 16)

        def fill_keys(nvec, ptr_vec):
            def kb(j, _):
                xv = buf[pl.ds(j * 16, 16)]
                valid = j < ptr_vec
                kv = jnp.where(valid, _keys16(xv), np.uint32(0))
                keybuf[pl.ds(j * 16, 16)] = kv
                return 0
            lax.fori_loop(0, nvec, kb, 0)

        def kth_key(nvec):
            d = d_ref[0]
            tprev = tkey_ref[0]

            def bit_step(b, acc):
                t = acc | (np.uint32(1) << (np.uint32(31) - b.astype(jnp.uint32)))

                def cstep(j, cv):
                    kv = keybuf[pl.ds(j * 16, 16)]
                    return cv + (kv >= t).astype(jnp.int32)
                cv = lax.fori_loop(0, nvec, cstep, jnp.zeros((16,), jnp.int32))
                cnt = jnp.sum(cv) + jnp.where(tprev >= t, d, 0)
                return jnp.where(cnt >= K, t, acc)
            return lax.fori_loop(0, 32, bit_step, jnp.zeros((), jnp.uint32))

        def compact(ptr_vec, tv):
            nvec = jnp.max(ptr_vec)
            fill_keys(nvec, ptr_vec)
            tkey = kth_key(nvec)
            new_tv = _unkey16(jnp.full((16,), tkey, jnp.uint32))

            def cstep(j, newp):
                kv = keybuf[pl.ds(j * 16, 16)]
                xv = buf[pl.ds(j * 16, 16)]
                m = kv > tkey
                idx = jnp.left_shift(newp, 4) | lanes
                plsc.store_scatter(buf, [idx], xv, mask=m)
                return newp + m.astype(jnp.int32)
            newp = lax.fori_loop(0, nvec, cstep, jnp.zeros((16,), jnp.int32))
            d_ref[0] = K - jnp.sum(newp)
            tkey_ref[0] = tkey
            return newp, new_tv

        def maybe_compact(ptr_vec, tv):
            return lax.cond(jnp.max(ptr_vec) > LTRIG, compact,
                            lambda p, t: (p, t), ptr_vec, tv)

        for r in range(ROWS_PER_W):
            row = wid * ROWS_PER_W + r
            d_ref[0] = 0
            tkey_ref[0] = jnp.zeros((), jnp.uint32)
            state = (jnp.zeros((16,), jnp.int32),
                     jnp.full((16,), -np.inf, jnp.float32))

            pltpu.make_async_copy(
                x_hbm.at[row * NCHUNK],
                chunks.at[pl.ds(0, CHUNK)], sems.at[0]).start()

            def chunk_step(i, state):
                slot = (i % 2) * CHUNK
                nslot = ((i + 1) % 2) * CHUNK

                @pl.when(i + 1 < NCHUNK)
                def _():
                    pltpu.make_async_copy(
                        x_hbm.at[row * NCHUNK + i + 1],
                        chunks.at[pl.ds(nslot, CHUNK)],
                        sems.at[(i + 1) % 2]).start()
                pltpu.make_async_copy(
                    x_hbm.at[row * NCHUNK + i],
                    chunks.at[pl.ds(slot, CHUNK)], sems.at[i % 2]).wait()
                chunk = chunks.at[pl.ds(slot, CHUNK)]

                def sub_step(s, state):
                    def vstep(v, state):
                        ptr_vec, tv = state
                        for u in range(UNROLL):
                            xv = chunk[pl.ds((s * SUBVEC + v * UNROLL + u) * 16, 16)]
                            m = xv > tv
                            idx = jnp.left_shift(ptr_vec, 4) | lanes
                            plsc.store_scatter(buf, [idx], xv, mask=m)
                            ptr_vec = ptr_vec + m.astype(jnp.int32)
                        return (ptr_vec, tv)
                    state = lax.fori_loop(0, SUBVEC // UNROLL, vstep, state)
                    return maybe_compact(*state)
                return lax.fori_loop(0, SUBS, sub_step, state)
            ptr_vec, tv = lax.fori_loop(0, NCHUNK, chunk_step, state)

            nvec = jnp.max(ptr_vec)
            fill_keys(nvec, ptr_vec)
            tkey = kth_key(nvec)
            tf = jnp.max(_unkey16(jnp.full((16,), tkey, jnp.uint32)))

            def sstep(j, cs):
                cv, sv = cs
                kv = keybuf[pl.ds(j * 16, 16)]
                xv = buf[pl.ds(j * 16, 16)]
                m = kv > tkey
                return (cv + m.astype(jnp.int32), sv + jnp.where(m, xv, 0.0))
            cv, sv = lax.fori_loop(0, nvec, sstep,
                                   (jnp.zeros((16,), jnp.int32),
                                    jnp.zeros((16,), jnp.float32)))
            c = jnp.sum(cv)
            s = jnp.sum(sv)
            mean = (s + (K - c).astype(jnp.float32) * tf) * np.float32(1.0 / K)
            if r == 0:
                outv[...] = jnp.where(lanes == 0, mean, 0.0)
            else:
                outv[...] = jnp.where(lanes == r, mean, outv[...])
        pltpu.sync_copy(outv, out_hbm.at[wid])

    return pl.kernel(
        body,
        mesh=mesh,
        compiler_params=pltpu.CompilerParams(needs_layout_passes=False),
        out_type=jax.ShapeDtypeStruct((NW, 16), jnp.float32),
        scratch_types=[
            pltpu.VMEM((2 * CHUNK,), jnp.float32),
            pltpu.VMEM((SUBCAP * 16,), jnp.float32),
            pltpu.VMEM((SUBCAP * 16,), jnp.uint32),
            pltpu.VMEM((16,), jnp.float32),
            pltpu.SMEM((1,), jnp.int32),
            pltpu.SMEM((1,), jnp.uint32),
            pltpu.SemaphoreType.DMA((2,)),
        ],
    )


_kernel_call_cache = []


def kernel(x):
    if not _kernel_call_cache:
        _kernel_call_cache.append(_mk_kernel())
    xf = x.reshape(ROWS * NCHUNK, CHUNK)
    out = _kernel_call_cache[0](xf)
    return out[:, :ROWS_PER_W].reshape(ROWS)


# speedup vs baseline: 50.6435x; 2.0264x over previous
import jax
import jax.numpy as jnp
import numpy as np
from jax import lax
from jax.experimental import pallas as pl
from jax.experimental.pallas import tpu as pltpu
from jax.experimental.pallas import tpu_sc as plsc

K = 256
NCORES = 2
NSUB = 16
NW = NCORES * NSUB
ROWS = 64
ROWS_PER_W = ROWS // NW
ROW_ELEMS = 32 * 32768
CHUNK = 16384
NCHUNK = ROW_ELEMS // CHUNK
SUBVEC = 256
SUBS = CHUNK // (SUBVEC * 16)
UNROLL = 16
SUBCAP = 512
LTRIG = 255

_SIGN = np.uint32(0x80000000)


def _keys16(xv):
    b = lax.bitcast_convert_type(xv, jnp.uint32)
    return jnp.where(b >= _SIGN, ~b, b | _SIGN)


def _unkey16(kv):
    b = jnp.where(kv >= _SIGN, kv & np.uint32(0x7FFFFFFF), ~kv)
    return lax.bitcast_convert_type(b, jnp.float32)


def _mk_kernel():
    mesh = plsc.VectorSubcoreMesh(
        core_axis_name="c", subcore_axis_name="s", num_cores=NCORES)

    def body(x_hbm, out_hbm, chunks, buf, keybuf, outv, d_ref, tkey_ref, sems):
        cid = lax.axis_index("c")
        sid = lax.axis_index("s")
        wid = sid * NCORES + cid
        lanes = lax.iota(jnp.int32, 16)

        def fill_keys(nvec, ptr_vec):
            def kb(j, _):
                xv = buf[pl.ds(j * 16, 16)]
                valid = j < ptr_vec
                kv = jnp.where(valid, _keys16(xv), np.uint32(0))
                keybuf[pl.ds(j * 16, 16)] = kv
                return 0
            lax.fori_loop(0, nvec, kb, 0)

        def kth_key(nvec):
            d = d_ref[0]
            tprev = tkey_ref[0]

            def bit_step(b, acc):
                t = acc | (np.uint32(1) << (np.uint32(31) - b.astype(jnp.uint32)))

                def cstep(j, cv):
                    kv = keybuf[pl.ds(j * 16, 16)]
                    return cv + (kv >= t).astype(jnp.int32)
                cv = lax.fori_loop(0, nvec, cstep, jnp.zeros((16,), jnp.int32))
                cnt = jnp.sum(cv) + jnp.where(tprev >= t, d, 0)
                return jnp.where(cnt >= K, t, acc)
            return lax.fori_loop(0, 32, bit_step, jnp.zeros((), jnp.uint32))

        def compact(ptr_vec, tv):
            nvec = jnp.max(ptr_vec)
            fill_keys(nvec, ptr_vec)
            tkey = kth_key(nvec)
            new_tv = _unkey16(jnp.full((16,), tkey, jnp.uint32))

            def cstep(j, newp):
                kv = keybuf[pl.ds(j * 16, 16)]
                xv = buf[pl.ds(j * 16, 16)]
                m = kv > tkey
                idx = jnp.left_shift(newp, 4) | lanes
                plsc.store_scatter(buf, [idx], xv, mask=m)
                return newp + m.astype(jnp.int32)
            newp = lax.fori_loop(0, nvec, cstep, jnp.zeros((16,), jnp.int32))
            d_ref[0] = K - jnp.sum(newp)
            tkey_ref[0] = tkey
            return newp, new_tv

        def maybe_compact(ptr_vec, tv):
            return lax.cond(jnp.max(ptr_vec) > LTRIG, compact,
                            lambda p, t: (p, t), ptr_vec, tv)

        for r in range(ROWS_PER_W):
            row = wid * ROWS_PER_W + r
            d_ref[0] = 0
            tkey_ref[0] = jnp.zeros((), jnp.uint32)
            state = (jnp.zeros((16,), jnp.int32),
                     jnp.full((16,), -np.inf, jnp.float32))

            pltpu.make_async_copy(
                x_hbm.at[row * NCHUNK],
                chunks.at[pl.ds(0, CHUNK)], sems.at[0]).start()

            def chunk_step(i, state):
                slot = (i % 2) * CHUNK
                nslot = ((i + 1) % 2) * CHUNK

                @pl.when(i + 1 < NCHUNK)
                def _():
                    pltpu.make_async_copy(
                        x_hbm.at[row * NCHUNK + i + 1],
                        chunks.at[pl.ds(nslot, CHUNK)],
                        sems.at[(i + 1) % 2]).start()
                pltpu.make_async_copy(
                    x_hbm.at[row * NCHUNK + i],
                    chunks.at[pl.ds(slot, CHUNK)], sems.at[i % 2]).wait()
                chunk = chunks.at[pl.ds(slot, CHUNK)]

                def sub_step(s, state):
                    @plsc.parallel_loop(0, SUBVEC, 1, unroll=UNROLL,
                                        carry=state)
                    def vstep(v, st):
                        ptr_vec, tv = st
                        xv = chunk[pl.ds((s * SUBVEC + v) * 16, 16)]
                        m = xv > tv
                        idx = jnp.left_shift(ptr_vec, 4) | lanes
                        plsc.store_scatter(buf, [idx], xv, mask=m)
                        return (ptr_vec + m.astype(jnp.int32), tv)
                    return maybe_compact(*vstep)
                return lax.fori_loop(0, SUBS, sub_step, state)
            ptr_vec, tv = lax.fori_loop(0, NCHUNK, chunk_step, state)

            nvec = jnp.max(ptr_vec)
            fill_keys(nvec, ptr_vec)
            tkey = kth_key(nvec)
            tf = jnp.max(_unkey16(jnp.full((16,), tkey, jnp.uint32)))

            def sstep(j, cs):
                cv, sv = cs
                kv = keybuf[pl.ds(j * 16, 16)]
                xv = buf[pl.ds(j * 16, 16)]
                m = kv > tkey
                return (cv + m.astype(jnp.int32), sv + jnp.where(m, xv, 0.0))
            cv, sv = lax.fori_loop(0, nvec, sstep,
                                   (jnp.zeros((16,), jnp.int32),
                                    jnp.zeros((16,), jnp.float32)))
            c = jnp.sum(cv)
            s = jnp.sum(sv)
            mean = (s + (K - c).astype(jnp.float32) * tf) * np.float32(1.0 / K)
            if r == 0:
                outv[...] = jnp.where(lanes == 0, mean, 0.0)
            else:
                outv[...] = jnp.where(lanes == r, mean, outv[...])
        pltpu.sync_copy(outv, out_hbm.at[wid])

    return pl.kernel(
        body,
        mesh=mesh,
        compiler_params=pltpu.CompilerParams(needs_layout_passes=False),
        out_type=jax.ShapeDtypeStruct((NW, 16), jnp.float32),
        scratch_types=[
            pltpu.VMEM((2 * CHUNK,), jnp.float32),
            pltpu.VMEM((SUBCAP * 16,), jnp.float32),
            pltpu.VMEM((SUBCAP * 16,), jnp.uint32),
            pltpu.VMEM((16,), jnp.float32),
            pltpu.SMEM((1,), jnp.int32),
            pltpu.SMEM((1,), jnp.uint32),
            pltpu.SemaphoreType.DMA((2,)),
        ],
    )


_kernel_call_cache = []


def kernel(x):
    if not _kernel_call_cache:
        _kernel_call_cache.append(_mk_kernel())
    xf = x.reshape(ROWS * NCHUNK, CHUNK)
    out = _kernel_call_cache[0](xf)
    return out[:, :ROWS_PER_W].reshape(ROWS)


# speedup vs baseline: 54.1570x; 1.0694x over previous
import jax
import jax.numpy as jnp
import numpy as np
from jax import lax
from jax.experimental import pallas as pl
from jax.experimental.pallas import tpu as pltpu
from jax.experimental.pallas import tpu_sc as plsc

K = 256
NCORES = 2
NSUB = 16
NW = NCORES * NSUB
ROWS = 64
ROWS_PER_W = ROWS // NW
ROW_ELEMS = 32 * 32768
CHUNK = 16384
NCHUNK = ROW_ELEMS // CHUNK
SUBVEC = 256
SUBS = CHUNK // (SUBVEC * 16)
UNROLL = 16
SUBCAP = 512
LTRIG = 255

_SIGN = np.uint32(0x80000000)


def _keys16(xv):
    b = lax.bitcast_convert_type(xv, jnp.uint32)
    return jnp.where(b >= _SIGN, ~b, b | _SIGN)


def _unkey16(kv):
    b = jnp.where(kv >= _SIGN, kv & np.uint32(0x7FFFFFFF), ~kv)
    return lax.bitcast_convert_type(b, jnp.float32)


def _mk_kernel():
    mesh = plsc.VectorSubcoreMesh(
        core_axis_name="c", subcore_axis_name="s", num_cores=NCORES)

    def body(x_hbm, out_hbm, chunks, buf, keybuf, outv, d_ref, tkey_ref, sems):
        cid = lax.axis_index("c")
        sid = lax.axis_index("s")
        wid = sid * NCORES + cid
        lanes = lax.iota(jnp.int32, 16)

        def fill_keys(nvec, ptr_vec):
            def kb(j, _):
                xv = buf[pl.ds(j * 16, 16)]
                valid = j < ptr_vec
                kv = jnp.where(valid, _keys16(xv), np.uint32(0))
                keybuf[pl.ds(j * 16, 16)] = kv
                return 0
            lax.fori_loop(0, nvec, kb, 0)

        def kth_key(nvec):
            d = d_ref[0]
            tprev = tkey_ref[0]

            def bit_step(b, acc):
                t = acc | (np.uint32(1) << (np.uint32(31) - b.astype(jnp.uint32)))

                def cstep(j, cv):
                    kv = keybuf[pl.ds(j * 16, 16)]
                    return cv + (kv >= t).astype(jnp.int32)
                cv = lax.fori_loop(0, nvec, cstep, jnp.zeros((16,), jnp.int32))
                cnt = jnp.sum(cv) + jnp.where(tprev >= t, d, 0)
                return jnp.where(cnt >= K, t, acc)
            return lax.fori_loop(0, 32, bit_step, jnp.zeros((), jnp.uint32))

        def compact(sptr, tv):
            ptr_vec = lax.shift_right_logical(sptr, 4)
            nvec = jnp.max(ptr_vec)
            fill_keys(nvec, ptr_vec)
            tkey = kth_key(nvec)
            new_tv = _unkey16(jnp.full((16,), tkey, jnp.uint32))

            def cstep(j, newsp):
                kv = keybuf[pl.ds(j * 16, 16)]
                xv = buf[pl.ds(j * 16, 16)]
                m = kv > tkey
                plsc.store_scatter(buf, [newsp], xv, mask=m)
                return newsp + jnp.where(m, 16, 0)
            newsp = lax.fori_loop(0, nvec, cstep, lanes)
            d_ref[0] = K - jnp.sum(lax.shift_right_logical(newsp, 4))
            tkey_ref[0] = tkey
            return newsp, new_tv

        def maybe_compact(sptr, tv):
            return lax.cond(jnp.max(sptr) > LTRIG * 16 + 16, compact,
                            lambda p, t: (p, t), sptr, tv)

        for r in range(ROWS_PER_W):
            row = wid * ROWS_PER_W + r
            d_ref[0] = 0
            tkey_ref[0] = jnp.zeros((), jnp.uint32)
            state = (lanes, jnp.full((16,), -np.inf, jnp.float32))

            pltpu.make_async_copy(
                x_hbm.at[row * NCHUNK],
                chunks.at[pl.ds(0, CHUNK)], sems.at[0]).start()

            def chunk_step(i, state):
                slot = (i % 2) * CHUNK
                nslot = ((i + 1) % 2) * CHUNK

                @pl.when(i + 1 < NCHUNK)
                def _():
                    pltpu.make_async_copy(
                        x_hbm.at[row * NCHUNK + i + 1],
                        chunks.at[pl.ds(nslot, CHUNK)],
                        sems.at[(i + 1) % 2]).start()
                pltpu.make_async_copy(
                    x_hbm.at[row * NCHUNK + i],
                    chunks.at[pl.ds(slot, CHUNK)], sems.at[i % 2]).wait()
                chunk = chunks.at[pl.ds(slot, CHUNK)]

                def sub_step(s, state):
                    @plsc.parallel_loop(0, SUBVEC, 1, unroll=UNROLL,
                                        carry=state)
                    def vstep(v, st):
                        sptr, tv = st
                        xv = chunk[pl.ds((s * SUBVEC + v) * 16, 16)]
                        m = xv > tv
                        plsc.store_scatter(buf, [sptr], xv, mask=m)
                        return (sptr + jnp.where(m, 16, 0), tv)
                    return maybe_compact(*vstep)
                return lax.fori_loop(0, SUBS, sub_step, state)
            sptr, tv = lax.fori_loop(0, NCHUNK, chunk_step, state)
            ptr_vec = lax.shift_right_logical(sptr, 4)

            nvec = jnp.max(ptr_vec)
            fill_keys(nvec, ptr_vec)
            tkey = kth_key(nvec)
            tf = jnp.max(_unkey16(jnp.full((16,), tkey, jnp.uint32)))

            def sstep(j, cs):
                cv, sv = cs
                kv = keybuf[pl.ds(j * 16, 16)]
                xv = buf[pl.ds(j * 16, 16)]
                m = kv > tkey
                return (cv + m.astype(jnp.int32), sv + jnp.where(m, xv, 0.0))
            cv, sv = lax.fori_loop(0, nvec, sstep,
                                   (jnp.zeros((16,), jnp.int32),
                                    jnp.zeros((16,), jnp.float32)))
            c = jnp.sum(cv)
            s = jnp.sum(sv)
            mean = (s + (K - c).astype(jnp.float32) * tf) * np.float32(1.0 / K)
            if r == 0:
                outv[...] = jnp.where(lanes == 0, mean, 0.0)
            else:
                outv[...] = jnp.where(lanes == r, mean, outv[...])
        pltpu.sync_copy(outv, out_hbm.at[wid])

    return pl.kernel(
        body,
        mesh=mesh,
        compiler_params=pltpu.CompilerParams(needs_layout_passes=False),
        out_type=jax.ShapeDtypeStruct((NW, 16), jnp.float32),
        scratch_types=[
            pltpu.VMEM((2 * CHUNK,), jnp.float32),
            pltpu.VMEM((SUBCAP * 16,), jnp.float32),
            pltpu.VMEM((SUBCAP * 16,), jnp.uint32),
            pltpu.VMEM((16,), jnp.float32),
            pltpu.SMEM((1,), jnp.int32),
            pltpu.SMEM((1,), jnp.uint32),
            pltpu.SemaphoreType.DMA((2,)),
        ],
    )


_kernel_call_cache = []


def kernel(x):
    if not _kernel_call_cache:
        _kernel_call_cache.append(_mk_kernel())
    xf = x.reshape(ROWS * NCHUNK, CHUNK)
    out = _kernel_call_cache[0](xf)
    return out[:, :ROWS_PER_W].reshape(ROWS)
